# Initial kernel scaffold; baseline (speedup 1.0000x reference)
import jax
import jax.numpy as jnp
from jax.experimental import pallas as pl


def kernel(z, pos_embed):
    raise NotImplementedError("write your pallas kernel here")


# speedup vs baseline: 1.4189x; 1.4189x over previous
import functools

import jax
import jax.numpy as jnp
from jax import lax
from jax.experimental import pallas as pl
from jax.experimental.pallas import tpu as pltpu
from jax.experimental.pallas import tpu_sc as plsc


def _const_embed_sc(pos_embed, batch_n):
    S, D = pos_embed.shape
    NC, NS = 2, 16
    NW = NC * NS
    rows_per_w = S // NW

    mesh = plsc.VectorSubcoreMesh(core_axis_name="c", subcore_axis_name="s")

    @functools.partial(
        pl.kernel,
        out_type=jax.ShapeDtypeStruct((S, batch_n, D), jnp.float32),
        mesh=mesh,
        scratch_types=[
            pltpu.VMEM((rows_per_w, D), jnp.float32),
            pltpu.SemaphoreType.DMA,
        ],
    )
    def k(pe_hbm, out_hbm, rows_v, sem):
        wid = lax.axis_index("s") * NC + lax.axis_index("c")
        base = wid * rows_per_w
        pltpu.sync_copy(pe_hbm.at[pl.ds(base, rows_per_w)], rows_v)
        copies = [
            pltpu.make_async_copy(
                rows_v, out_hbm.at[pl.ds(base, rows_per_w), n], sem
            )
            for n in range(batch_n)
        ]
        for c in copies:
            c.start()
        for c in copies:
            c.wait()

    return k(pos_embed)


def kernel(z, pos_embed):
    return _const_embed_sc(pos_embed, z.shape[1])
